# baseline (speedup 1.0000x reference)
import dataclasses
import functools

import jax
import jax.numpy as jnp
from jax import lax
from jax.experimental import pallas as pl
from jax.experimental.pallas import tpu as pltpu
from jax.experimental.pallas import tpu_sc as plsc

N = 10000
E = 320000
H = 78
HP = 128
NTILES = 32
EP = E // NTILES
CH = 80
NCH = EP // CH
NG = CH // 16
RB = 624
EB = 16000

_f32 = jnp.float32
_mesh = plsc.VectorSubcoreMesh(core_axis_name="c", subcore_axis_name="s")

_sc_params = pltpu.CompilerParams()
if "needs_layout_passes" in pltpu.CompilerParams.__dataclass_fields__:
    _sc_params = dataclasses.replace(_sc_params, needs_layout_passes=False)


def _tc1_body(x_ref, w_ref, asw_ref, adw_ref, htab_ref, as_ref, ad_ref):
    h = jnp.dot(x_ref[...], w_ref[...], preferred_element_type=_f32)
    onehot = (lax.broadcasted_iota(jnp.int32, (1, HP), 1) == H).astype(_f32)
    htab_ref[...] = h + onehot
    as_ref[...] = jnp.dot(h, asw_ref[...], preferred_element_type=_f32)
    ad_ref[...] = jnp.dot(h, adw_ref[...], preferred_element_type=_f32)


def _tc2_body(acc_ref, b_ref, w_ref, asw_ref, adw_ref,
              htab_ref, as_ref, ad_ref):
    accs = acc_ref[0] + acc_ref[1]
    denom = accs[:, H:H + 1]
    z = jnp.maximum(accs / (denom + 1e-16) + b_ref[...], 0.0)
    h = jnp.dot(z, w_ref[...], preferred_element_type=_f32)
    onehot = (lax.broadcasted_iota(jnp.int32, (1, HP), 1) == H).astype(_f32)
    htab_ref[...] = h + onehot
    as_ref[...] = jnp.dot(h, asw_ref[...], preferred_element_type=_f32)
    ad_ref[...] = jnp.dot(h, adw_ref[...], preferred_element_type=_f32)


def _tc3_body(acc_ref, b_ref, wu_ref, wv_ref, u_ref, v_ref):
    accs = acc_ref[0] + acc_ref[1]
    denom = accs[:, H:H + 1]
    hf = accs / (denom + 1e-16) + b_ref[...]
    u_ref[...] = jnp.dot(hf, wu_ref[...], preferred_element_type=_f32)
    v_ref[...] = jnp.dot(hf, wv_ref[...], preferred_element_type=_f32)


def _tce_body(ea_ref, wm1_ref, bm1_ref, wm2_ref, bm2_ref, wfc_ref, bf_ref,
              w_ref):
    t = jnp.dot(ea_ref[...], wm1_ref[...], preferred_element_type=_f32)
    t = jnp.maximum(t + bm1_ref[...], 0.0)
    wv = jnp.dot(wm2_ref[...], wfc_ref[...], preferred_element_type=_f32)
    c0 = jnp.dot(bm2_ref[...], wfc_ref[...], preferred_element_type=_f32)
    w_ref[...] = jnp.dot(t, wv, preferred_element_type=_f32) + c0 + bf_ref[...]


NBUF = 4


def _sc_gat(htab, aa, pidx):
    @functools.partial(
        pl.kernel,
        out_type=jax.ShapeDtypeStruct((2, N, HP), _f32),
        mesh=_mesh,
        compiler_params=_sc_params,
        scratch_types=(
            [pltpu.VMEM((CH, HP), _f32)] * NBUF
            + [pltpu.VMEM((4 * CH,), jnp.int32)] * NBUF
            + [pltpu.VMEM((CH,), jnp.int32)] * NBUF
            + [pltpu.VMEM((2 * CH,), _f32)] * NBUF
            + [pltpu.SemaphoreType.DMA] * (3 * NBUF)
            + [pltpu.VMEM_SHARED((N, HP), _f32)]
        ),
    )
    def k(htab_hbm, aa_hbm, pidx_hbm, out_hbm, *scratch):
        rows = scratch[0:NBUF]
        pidx = scratch[NBUF:2 * NBUF]
        didxs = scratch[2 * NBUF:3 * NBUF]
        aag = scratch[3 * NBUF:4 * NBUF]
        isem = scratch[4 * NBUF:5 * NBUF]
        gsem = scratch[5 * NBUF:6 * NBUF]
        ssem = scratch[6 * NBUF:7 * NBUF]
        acc_sh = scratch[7 * NBUF]

        cid = lax.axis_index("c")
        sid = lax.axis_index("s")
        wid = sid * 2 + cid

        def fire_idx(c, b):
            off = (wid * NCH + c) * 4 * CH
            pltpu.async_copy(pidx_hbm.at[pl.ds(off, 4 * CH)], pidx[b],
                             isem[b])

        def wait_idx(c, b):
            off = (wid * NCH + c) * 4 * CH
            pltpu.make_async_copy(pidx_hbm.at[pl.ds(off, 4 * CH)], pidx[b],
                                  isem[b]).wait()

        HC = CH // 2

        def fire_gather(b):
            pltpu.async_copy(htab_hbm.at[pidx[b].at[pl.ds(0, HC)]],
                             rows[b].at[pl.ds(0, HC)], gsem[b])
            pltpu.async_copy(htab_hbm.at[pidx[b].at[pl.ds(HC, HC)]],
                             rows[b].at[pl.ds(HC, HC)], gsem[b])
            pltpu.async_copy(aa_hbm.at[pidx[b].at[pl.ds(2 * CH, CH)]],
                             aag[b].at[pl.ds(0, CH)], gsem[b])
            pltpu.async_copy(aa_hbm.at[pidx[b].at[pl.ds(3 * CH, CH)]],
                             aag[b].at[pl.ds(CH, CH)], gsem[b])

        def wait_gather(b):
            pltpu.make_async_copy(htab_hbm.at[pidx[b].at[pl.ds(0, HC)]],
                                  rows[b].at[pl.ds(0, HC)], gsem[b]).wait()
            pltpu.make_async_copy(htab_hbm.at[pidx[b].at[pl.ds(HC, HC)]],
                                  rows[b].at[pl.ds(HC, HC)], gsem[b]).wait()
            pltpu.make_async_copy(aa_hbm.at[pidx[b].at[pl.ds(2 * CH, CH)]],
                                  aag[b].at[pl.ds(0, CH)], gsem[b]).wait()
            pltpu.make_async_copy(aa_hbm.at[pidx[b].at[pl.ds(3 * CH, CH)]],
                                  aag[b].at[pl.ds(CH, CH)], gsem[b]).wait()

        def fire_scatter(b):
            pltpu.async_copy(rows[b], acc_sh.at[didxs[b]], ssem[b],
                             add=True)

        def wait_scatter(b):
            pltpu.make_async_copy(rows[b], acc_sh.at[didxs[b]],
                                  ssem[b]).wait()

        def body(c, b, bp, do_fire_idx, do_fire_gather, do_wait_scatter):
            wait_gather(b)

            @pl.loop(0, NG)
            def _scale(g):
                e16 = (aag[b][pl.ds(g * 16, 16)]
                       + aag[b][pl.ds(CH + g * 16, 16)])
                p16 = jnp.exp(jnp.maximum(e16, 0.2 * e16))
                didxs[b][pl.ds(g * 16, 16)] = pidx[b][pl.ds(CH + g * 16, 16)]
                for j in range(16):
                    prv = jnp.full((16,), p16[j], _f32)
                    r = g * 16 + j
                    for cc in range(5):
                        rows[b][r, pl.ds(cc * 16, 16)] = (
                            rows[b][r, pl.ds(cc * 16, 16)] * prv)

            fire_scatter(b)
            if do_fire_idx:
                fire_idx(c + NBUF, b)
            if do_fire_gather:
                if do_wait_scatter:
                    wait_scatter(bp)
                wait_idx(c + 2, bp)
                fire_gather(bp)

        zero16 = jnp.zeros((16,), _f32)

        @pl.loop(0, CH)
        def _zero(r):
            for cc in range(HP // 16):
                rows[0][r, pl.ds(cc * 16, 16)] = zero16

        base = sid * RB

        @pl.loop(0, RB // CH)
        def _init(i):
            pltpu.sync_copy(rows[0], acc_sh.at[pl.ds(base + i * CH, CH)])

        pltpu.sync_copy(rows[0].at[pl.ds(0, RB - (RB // CH) * CH)],
                        acc_sh.at[pl.ds(base + (RB // CH) * CH,
                                        RB - (RB // CH) * CH)])

        @pl.when(sid == 15)
        def _init_tail():
            pltpu.sync_copy(rows[0].at[pl.ds(0, N - 16 * RB)],
                            acc_sh.at[pl.ds(16 * RB, N - 16 * RB)])

        plsc.subcore_barrier()

        for b in range(NBUF):
            fire_idx(b, b)
        wait_idx(0, 0)
        fire_gather(0)
        wait_idx(1, 1)
        fire_gather(1)

        body(0, 0, 2, True, True, False)
        body(1, 1, 3, True, True, False)

        @pl.loop(0, (NCH - 9) // NBUF)
        def _steady(it):
            c0 = 2 + NBUF * it
            for kk in range(NBUF):
                body(c0 + kk, (2 + kk) % NBUF, kk, True, True, True)

        for c in range(NCH - 7, NCH):
            body(c, c % NBUF, (c + 2) % NBUF,
                 c + NBUF < NCH, c + 2 < NCH, True)

        for c in range(NCH - 4, NCH):
            wait_scatter(c % NBUF)

        plsc.subcore_barrier()
        pltpu.sync_copy(acc_sh.at[pl.ds(base, RB)],
                        out_hbm.at[cid, pl.ds(base, RB)])

        @pl.when(sid == 15)
        def _out_tail():
            pltpu.sync_copy(acc_sh.at[pl.ds(16 * RB, N - 16 * RB)],
                            out_hbm.at[cid, pl.ds(16 * RB, N - 16 * RB)])

    return k(htab, aa, pidx)


def _sc_final(u, v, w, src2, dst2):
    @functools.partial(
        pl.kernel,
        out_type=jax.ShapeDtypeStruct((E,), _f32),
        mesh=_mesh,
        compiler_params=_sc_params,
        scratch_types=[
            pltpu.VMEM((EP,), jnp.int32),
            pltpu.VMEM((EP,), jnp.int32),
            pltpu.VMEM((N,), _f32),
            pltpu.VMEM((N,), _f32),
            pltpu.VMEM((EP,), _f32),
            pltpu.VMEM((EP,), _f32),
        ],
    )
    def k(u_hbm, v_hbm, w_hbm, src_hbm, dst_hbm, out_hbm,
          src_v, dst_v, u_v, v_v, w_v, pred_v):
        cid = lax.axis_index("c")
        sid = lax.axis_index("s")
        wid = sid * 2 + cid

        pltpu.sync_copy(u_hbm, u_v)
        pltpu.sync_copy(v_hbm, v_v)
        pltpu.sync_copy(src_hbm.at[wid], src_v)
        pltpu.sync_copy(dst_hbm.at[wid], dst_v)
        pltpu.sync_copy(w_hbm.at[wid], w_v)

        @pl.loop(0, EP // 16)
        def _grp(g):
            s16 = src_v[pl.ds(g * 16, 16)]
            d16 = dst_v[pl.ds(g * 16, 16)]
            pred_v[pl.ds(g * 16, 16)] = (
                plsc.load_gather(u_v, [s16])
                + plsc.load_gather(v_v, [d16])
                + w_v[pl.ds(g * 16, 16)])

        pltpu.sync_copy(pred_v, out_hbm.at[pl.ds(wid * EP, EP)])

    return k(u, v, w, src2, dst2)


def kernel(x, edge_index, edge_attr, W1, a_s1, a_d1, b1,
           W2, a_s2, a_d2, b2, Wm1, bm1, Wm2, bm2, Wf, bf):
    src = edge_index[0].astype(jnp.int32)
    dst = edge_index[1].astype(jnp.int32)
    src2 = src.reshape(NTILES, EP)
    dst2 = dst.reshape(NTILES, EP)
    srcc = src.reshape(-1, CH)
    dstc = dst.reshape(-1, CH)
    pidx = jnp.concatenate([srcc, dstc, srcc, dstc + N],
                           axis=1).reshape(-1)

    pad2 = ((0, 0), (0, HP - H))
    W1p = jnp.pad(W1, pad2)
    W2p = jnp.pad(W2, ((0, HP - H), (0, HP - H)))
    asw1 = jnp.pad(a_s1, (0, HP - H)).reshape(HP, 1)
    adw1 = jnp.pad(a_d1, (0, HP - H)).reshape(HP, 1)
    asw2 = jnp.pad(a_s2, (0, HP - H)).reshape(HP, 1)
    adw2 = jnp.pad(a_d2, (0, HP - H)).reshape(HP, 1)
    b1p = jnp.pad(b1, (0, HP - H)).reshape(1, HP)
    b2p = jnp.pad(b2, (0, HP - H)).reshape(1, HP)
    wu = jnp.pad(Wf[:H, 0], (0, HP - H)).reshape(HP, 1)
    wv = jnp.pad(Wf[H:2 * H, 0], (0, HP - H)).reshape(HP, 1)
    wfc = Wf[2 * H:, :]
    bm1r = bm1.reshape(1, H)
    bm2r = bm2.reshape(1, H)
    bfr = bf.reshape(1, 1)

    htab1, as1, ad1 = pl.pallas_call(
        _tc1_body,
        out_shape=[
            jax.ShapeDtypeStruct((N, HP), _f32),
            jax.ShapeDtypeStruct((N, 1), _f32),
            jax.ShapeDtypeStruct((N, 1), _f32),
        ],
    )(x, W1p, asw1, adw1)

    w_edge = pl.pallas_call(
        _tce_body,
        grid=(E // EB,),
        in_specs=[
            pl.BlockSpec((EB, Wm1.shape[0]), lambda i: (i, 0)),
            pl.BlockSpec(Wm1.shape, lambda i: (0, 0)),
            pl.BlockSpec((1, H), lambda i: (0, 0)),
            pl.BlockSpec(Wm2.shape, lambda i: (0, 0)),
            pl.BlockSpec((1, H), lambda i: (0, 0)),
            pl.BlockSpec((H, 1), lambda i: (0, 0)),
            pl.BlockSpec((1, 1), lambda i: (0, 0)),
        ],
        out_specs=pl.BlockSpec((EB, 1), lambda i: (i, 0)),
        out_shape=jax.ShapeDtypeStruct((E, 1), _f32),
    )(edge_attr, Wm1, bm1r, Wm2, bm2r, wfc, bfr)

    aa1 = jnp.concatenate([as1.reshape(N), ad1.reshape(N)])
    acc1 = _sc_gat(htab1, aa1, pidx)

    htab2, as2, ad2 = pl.pallas_call(
        _tc2_body,
        out_shape=[
            jax.ShapeDtypeStruct((N, HP), _f32),
            jax.ShapeDtypeStruct((N, 1), _f32),
            jax.ShapeDtypeStruct((N, 1), _f32),
        ],
    )(acc1, b1p, W2p, asw2, adw2)

    aa2 = jnp.concatenate([as2.reshape(N), ad2.reshape(N)])
    acc2 = _sc_gat(htab2, aa2, pidx)

    u, v = pl.pallas_call(
        _tc3_body,
        out_shape=[
            jax.ShapeDtypeStruct((N, 1), _f32),
            jax.ShapeDtypeStruct((N, 1), _f32),
        ],
    )(acc2, b2p, wu, wv)

    pred = _sc_final(u.reshape(N), v.reshape(N),
                     w_edge.reshape(NTILES, EP), src2, dst2)
    return pred


# speedup vs baseline: 1.0570x; 1.0570x over previous
import dataclasses
import functools

import jax
import jax.numpy as jnp
from jax import lax
from jax.experimental import pallas as pl
from jax.experimental.pallas import tpu as pltpu
from jax.experimental.pallas import tpu_sc as plsc

N = 10000
E = 320000
H = 78
HP = 128
NTILES = 32
EP = E // NTILES
CH = 80
NCH = EP // CH
NG = CH // 16
RB = 624
EB = 16000

_f32 = jnp.float32
_mesh = plsc.VectorSubcoreMesh(core_axis_name="c", subcore_axis_name="s")

_sc_params = pltpu.CompilerParams()
if "needs_layout_passes" in pltpu.CompilerParams.__dataclass_fields__:
    _sc_params = dataclasses.replace(_sc_params, needs_layout_passes=False)


def _tc1_body(x_ref, w_ref, asw_ref, adw_ref, htab_ref, as_ref, ad_ref):
    h = jnp.dot(x_ref[...], w_ref[...], preferred_element_type=_f32)
    onehot = (lax.broadcasted_iota(jnp.int32, (1, HP), 1) == H).astype(_f32)
    htab_ref[...] = h + onehot
    as_ref[...] = jnp.dot(h, asw_ref[...], preferred_element_type=_f32)
    ad_ref[...] = jnp.dot(h, adw_ref[...], preferred_element_type=_f32)


def _tc2_body(acc_ref, b_ref, w_ref, asw_ref, adw_ref,
              htab_ref, as_ref, ad_ref):
    accs = acc_ref[0] + acc_ref[1]
    denom = accs[:, H:H + 1]
    z = jnp.maximum(accs / (denom + 1e-16) + b_ref[...], 0.0)
    h = jnp.dot(z, w_ref[...], preferred_element_type=_f32)
    onehot = (lax.broadcasted_iota(jnp.int32, (1, HP), 1) == H).astype(_f32)
    htab_ref[...] = h + onehot
    as_ref[...] = jnp.dot(h, asw_ref[...], preferred_element_type=_f32)
    ad_ref[...] = jnp.dot(h, adw_ref[...], preferred_element_type=_f32)


def _tc3_body(acc_ref, b_ref, wu_ref, wv_ref, u_ref, v_ref):
    accs = acc_ref[0] + acc_ref[1]
    denom = accs[:, H:H + 1]
    hf = accs / (denom + 1e-16) + b_ref[...]
    u_ref[...] = jnp.dot(hf, wu_ref[...], preferred_element_type=_f32)
    v_ref[...] = jnp.dot(hf, wv_ref[...], preferred_element_type=_f32)


def _tce_body(ea_ref, wm1_ref, bm1_ref, wm2_ref, bm2_ref, wfc_ref, bf_ref,
              w_ref):
    t = jnp.dot(ea_ref[...], wm1_ref[...], preferred_element_type=_f32)
    t = jnp.maximum(t + bm1_ref[...], 0.0)
    wv = jnp.dot(wm2_ref[...], wfc_ref[...], preferred_element_type=_f32)
    c0 = jnp.dot(bm2_ref[...], wfc_ref[...], preferred_element_type=_f32)
    w_ref[...] = jnp.dot(t, wv, preferred_element_type=_f32) + c0 + bf_ref[...]


NBUF = 4


def _sc_gat(htab, asn, adn, srcf, dstf):
    @functools.partial(
        pl.kernel,
        out_type=jax.ShapeDtypeStruct((2, N, HP), _f32),
        mesh=_mesh,
        compiler_params=_sc_params,
        scratch_types=(
            [pltpu.VMEM((CH, HP), _f32)] * NBUF
            + [pltpu.VMEM((CH,), jnp.int32)] * NBUF
            + [pltpu.VMEM((CH,), jnp.int32)] * NBUF
            + [pltpu.VMEM((CH,), jnp.int32)] * NBUF
            + [pltpu.VMEM((CH,), _f32)] * NBUF
            + [pltpu.VMEM((CH,), _f32)] * NBUF
            + [pltpu.SemaphoreType.DMA] * (3 * NBUF)
            + [pltpu.VMEM_SHARED((N, HP), _f32)]
        ),
    )
    def k(htab_hbm, as_hbm, ad_hbm, src_hbm, dst_hbm, out_hbm, *scratch):
        rows = scratch[0:NBUF]
        sidx = scratch[NBUF:2 * NBUF]
        didx = scratch[2 * NBUF:3 * NBUF]
        didxs = scratch[3 * NBUF:4 * NBUF]
        asg = scratch[4 * NBUF:5 * NBUF]
        adg = scratch[5 * NBUF:6 * NBUF]
        isem = scratch[6 * NBUF:7 * NBUF]
        gsem = scratch[7 * NBUF:8 * NBUF]
        ssem = scratch[8 * NBUF:9 * NBUF]
        acc_sh = scratch[9 * NBUF]

        cid = lax.axis_index("c")
        sid = lax.axis_index("s")
        wid = sid * 2 + cid

        def fire_idx(c, b):
            off = (wid * NCH + c) * CH
            pltpu.async_copy(src_hbm.at[pl.ds(off, CH)], sidx[b], isem[b])
            pltpu.async_copy(dst_hbm.at[pl.ds(off, CH)], didx[b], isem[b])

        def wait_idx(c, b):
            off = (wid * NCH + c) * CH
            pltpu.make_async_copy(src_hbm.at[pl.ds(off, CH)], sidx[b],
                                  isem[b]).wait()
            pltpu.make_async_copy(dst_hbm.at[pl.ds(off, CH)], didx[b],
                                  isem[b]).wait()

        def fire_gather(b):
            pltpu.async_copy(htab_hbm.at[sidx[b]], rows[b], gsem[b])
            pltpu.async_copy(as_hbm.at[sidx[b]], asg[b], gsem[b])
            pltpu.async_copy(ad_hbm.at[didx[b]], adg[b], gsem[b])

        def wait_gather(b):
            pltpu.make_async_copy(htab_hbm.at[sidx[b]], rows[b],
                                  gsem[b]).wait()
            pltpu.make_async_copy(as_hbm.at[sidx[b]], asg[b],
                                  gsem[b]).wait()
            pltpu.make_async_copy(ad_hbm.at[didx[b]], adg[b],
                                  gsem[b]).wait()

        def fire_scatter(b):
            pltpu.async_copy(rows[b], acc_sh.at[didxs[b]], ssem[b],
                             add=True)

        def wait_scatter(b):
            pltpu.make_async_copy(rows[b], acc_sh.at[didxs[b]],
                                  ssem[b]).wait()

        def body(c, b, bp, do_fire_idx, do_fire_gather, do_wait_scatter):
            wait_gather(b)

            @pl.loop(0, NG)
            def _scale(g):
                e16 = (asg[b][pl.ds(g * 16, 16)]
                       + adg[b][pl.ds(g * 16, 16)])
                p16 = jnp.exp(jnp.maximum(e16, 0.2 * e16))
                didxs[b][pl.ds(g * 16, 16)] = didx[b][pl.ds(g * 16, 16)]
                for j in range(16):
                    prv = jnp.full((16,), p16[j], _f32)
                    r = g * 16 + j
                    for cc in range(5):
                        rows[b][r, pl.ds(cc * 16, 16)] = (
                            rows[b][r, pl.ds(cc * 16, 16)] * prv)

            fire_scatter(b)
            if do_fire_idx:
                fire_idx(c + NBUF, b)
            if do_fire_gather:
                if do_wait_scatter:
                    wait_scatter(bp)
                wait_idx(c + 2, bp)
                fire_gather(bp)

        zero16 = jnp.zeros((16,), _f32)

        @pl.loop(0, CH)
        def _zero(r):
            for cc in range(HP // 16):
                rows[0][r, pl.ds(cc * 16, 16)] = zero16

        base = sid * RB

        @pl.loop(0, RB // CH)
        def _init(i):
            pltpu.sync_copy(rows[0], acc_sh.at[pl.ds(base + i * CH, CH)])

        pltpu.sync_copy(rows[0].at[pl.ds(0, RB - (RB // CH) * CH)],
                        acc_sh.at[pl.ds(base + (RB // CH) * CH,
                                        RB - (RB // CH) * CH)])

        @pl.when(sid == 15)
        def _init_tail():
            pltpu.sync_copy(rows[0].at[pl.ds(0, N - 16 * RB)],
                            acc_sh.at[pl.ds(16 * RB, N - 16 * RB)])

        plsc.subcore_barrier()

        for b in range(NBUF):
            fire_idx(b, b)
        wait_idx(0, 0)
        fire_gather(0)
        wait_idx(1, 1)
        fire_gather(1)

        body(0, 0, 2, True, True, False)
        body(1, 1, 3, True, True, False)

        @pl.loop(0, (NCH - 9) // NBUF)
        def _steady(it):
            c0 = 2 + NBUF * it
            for kk in range(NBUF):
                body(c0 + kk, (2 + kk) % NBUF, kk, True, True, True)

        for c in range(NCH - 7, NCH):
            body(c, c % NBUF, (c + 2) % NBUF,
                 c + NBUF < NCH, c + 2 < NCH, True)

        for c in range(NCH - 4, NCH):
            wait_scatter(c % NBUF)

        plsc.subcore_barrier()
        pltpu.sync_copy(acc_sh.at[pl.ds(base, RB)],
                        out_hbm.at[cid, pl.ds(base, RB)])

        @pl.when(sid == 15)
        def _out_tail():
            pltpu.sync_copy(acc_sh.at[pl.ds(16 * RB, N - 16 * RB)],
                            out_hbm.at[cid, pl.ds(16 * RB, N - 16 * RB)])

    return k(htab, asn, adn, srcf, dstf)


def _sc_final(u, v, w, src2, dst2):
    @functools.partial(
        pl.kernel,
        out_type=jax.ShapeDtypeStruct((E,), _f32),
        mesh=_mesh,
        compiler_params=_sc_params,
        scratch_types=[
            pltpu.VMEM((EP,), jnp.int32),
            pltpu.VMEM((EP,), jnp.int32),
            pltpu.VMEM((N,), _f32),
            pltpu.VMEM((N,), _f32),
            pltpu.VMEM((EP,), _f32),
            pltpu.VMEM((EP,), _f32),
        ],
    )
    def k(u_hbm, v_hbm, w_hbm, src_hbm, dst_hbm, out_hbm,
          src_v, dst_v, u_v, v_v, w_v, pred_v):
        cid = lax.axis_index("c")
        sid = lax.axis_index("s")
        wid = sid * 2 + cid

        pltpu.sync_copy(u_hbm, u_v)
        pltpu.sync_copy(v_hbm, v_v)
        pltpu.sync_copy(src_hbm.at[wid], src_v)
        pltpu.sync_copy(dst_hbm.at[wid], dst_v)
        pltpu.sync_copy(w_hbm.at[wid], w_v)

        @pl.loop(0, EP // 16)
        def _grp(g):
            s16 = src_v[pl.ds(g * 16, 16)]
            d16 = dst_v[pl.ds(g * 16, 16)]
            pred_v[pl.ds(g * 16, 16)] = (
                plsc.load_gather(u_v, [s16])
                + plsc.load_gather(v_v, [d16])
                + w_v[pl.ds(g * 16, 16)])

        pltpu.sync_copy(pred_v, out_hbm.at[pl.ds(wid * EP, EP)])

    return k(u, v, w, src2, dst2)


def kernel(x, edge_index, edge_attr, W1, a_s1, a_d1, b1,
           W2, a_s2, a_d2, b2, Wm1, bm1, Wm2, bm2, Wf, bf):
    src = edge_index[0].astype(jnp.int32)
    dst = edge_index[1].astype(jnp.int32)
    src2 = src.reshape(NTILES, EP)
    dst2 = dst.reshape(NTILES, EP)

    pad2 = ((0, 0), (0, HP - H))
    W1p = jnp.pad(W1, pad2)
    W2p = jnp.pad(W2, ((0, HP - H), (0, HP - H)))
    asw1 = jnp.pad(a_s1, (0, HP - H)).reshape(HP, 1)
    adw1 = jnp.pad(a_d1, (0, HP - H)).reshape(HP, 1)
    asw2 = jnp.pad(a_s2, (0, HP - H)).reshape(HP, 1)
    adw2 = jnp.pad(a_d2, (0, HP - H)).reshape(HP, 1)
    b1p = jnp.pad(b1, (0, HP - H)).reshape(1, HP)
    b2p = jnp.pad(b2, (0, HP - H)).reshape(1, HP)
    wu = jnp.pad(Wf[:H, 0], (0, HP - H)).reshape(HP, 1)
    wv = jnp.pad(Wf[H:2 * H, 0], (0, HP - H)).reshape(HP, 1)
    wfc = Wf[2 * H:, :]
    bm1r = bm1.reshape(1, H)
    bm2r = bm2.reshape(1, H)
    bfr = bf.reshape(1, 1)

    htab1, as1, ad1 = pl.pallas_call(
        _tc1_body,
        out_shape=[
            jax.ShapeDtypeStruct((N, HP), _f32),
            jax.ShapeDtypeStruct((N, 1), _f32),
            jax.ShapeDtypeStruct((N, 1), _f32),
        ],
    )(x, W1p, asw1, adw1)

    w_edge = pl.pallas_call(
        _tce_body,
        grid=(E // EB,),
        in_specs=[
            pl.BlockSpec((EB, Wm1.shape[0]), lambda i: (i, 0)),
            pl.BlockSpec(Wm1.shape, lambda i: (0, 0)),
            pl.BlockSpec((1, H), lambda i: (0, 0)),
            pl.BlockSpec(Wm2.shape, lambda i: (0, 0)),
            pl.BlockSpec((1, H), lambda i: (0, 0)),
            pl.BlockSpec((H, 1), lambda i: (0, 0)),
            pl.BlockSpec((1, 1), lambda i: (0, 0)),
        ],
        out_specs=pl.BlockSpec((EB, 1), lambda i: (i, 0)),
        out_shape=jax.ShapeDtypeStruct((E, 1), _f32),
    )(edge_attr, Wm1, bm1r, Wm2, bm2r, wfc, bfr)

    acc1 = _sc_gat(htab1, as1.reshape(N), ad1.reshape(N), src, dst)

    htab2, as2, ad2 = pl.pallas_call(
        _tc2_body,
        out_shape=[
            jax.ShapeDtypeStruct((N, HP), _f32),
            jax.ShapeDtypeStruct((N, 1), _f32),
            jax.ShapeDtypeStruct((N, 1), _f32),
        ],
    )(acc1, b1p, W2p, asw2, adw2)

    acc2 = _sc_gat(htab2, as2.reshape(N), ad2.reshape(N), src, dst)

    u, v = pl.pallas_call(
        _tc3_body,
        out_shape=[
            jax.ShapeDtypeStruct((N, 1), _f32),
            jax.ShapeDtypeStruct((N, 1), _f32),
        ],
    )(acc2, b2p, wu, wv)

    pred = _sc_final(u.reshape(N), v.reshape(N),
                     w_edge.reshape(NTILES, EP), src2, dst2)
    return pred
